# baseline (speedup 1.0000x reference)
import functools

import jax
import jax.numpy as jnp
from jax import lax
from jax.experimental import pallas as pl
from jax.experimental.pallas import tpu as pltpu
from jax.experimental.pallas import tpu_sc as plsc

B, S, H = 4, 8192, 1024
ROWS = B * S
NC, NS, L = 2, 16, 16
NW = NC * NS
ROWS_PER_W = ROWS // NW
CHUNK = 8
NCHUNKS = ROWS_PER_W // CHUNK
NB = 7
D = 5
VREGS_PER_ROW = H // L


def _sc_body(x_hbm, pos_hbm, tab_hbm, out_hbm, idx_v, ebuf, xobuf, *sems):
    gsems = sems[:NB]
    xsems = sems[NB:2 * NB]
    osems = sems[2 * NB:]
    wid = lax.axis_index("s") * NC + lax.axis_index("c")
    base = wid * ROWS_PER_W
    pltpu.sync_copy(pos_hbm.at[pl.ds(base, ROWS_PER_W)], idx_v)

    def fetch(n, b):
        idx_slice = idx_v.at[pl.ds(n * CHUNK, CHUNK)]
        pltpu.async_copy(tab_hbm.at[idx_slice], ebuf.at[b], gsems[b])
        pltpu.async_copy(x_hbm.at[pl.ds(base + n * CHUNK, CHUNK)],
                         xobuf.at[b], xsems[b])

    for n in range(D):
        fetch(n, n)

    T_OUTER = (NCHUNKS + NB - 1) // NB

    def outer(t, carry):
        for b in range(NB):
            s = t * NB + b

            n = s + D
            b_n = (b + D) % NB

            @pl.when(n < NCHUNKS)
            def _():
                @pl.when(n >= NB)
                def _():
                    pltpu.make_async_copy(
                        xobuf.at[b_n],
                        out_hbm.at[pl.ds(base + (n - NB) * CHUNK, CHUNK)],
                        osems[b_n]).wait()

                fetch(n, b_n)

            @pl.when(s < NCHUNKS)
            def _():
                rb = base + s * CHUNK
                idx_slice = idx_v.at[pl.ds(s * CHUNK, CHUNK)]
                pltpu.make_async_copy(tab_hbm.at[idx_slice], ebuf.at[b],
                                      gsems[b]).wait()
                pltpu.make_async_copy(x_hbm.at[pl.ds(rb, CHUNK)], xobuf.at[b],
                                      xsems[b]).wait()

                def row_body(r, c2):
                    for c in range(VREGS_PER_ROW):
                        sl = pl.ds(c * L, L)
                        plsc.addupdate(xobuf.at[b, r, sl], ebuf[b, r, sl])
                    return c2

                lax.fori_loop(0, CHUNK, row_body, 0)
                pltpu.async_copy(xobuf.at[b], out_hbm.at[pl.ds(rb, CHUNK)],
                                 osems[b])
        return carry

    lax.fori_loop(0, T_OUTER, outer, 0)

    for c in range(NCHUNKS - NB, NCHUNKS):
        b = c % NB
        pltpu.make_async_copy(xobuf.at[b],
                              out_hbm.at[pl.ds(base + c * CHUNK, CHUNK)],
                              osems[b]).wait()


@jax.jit
def kernel(x, input_pos, position_embeddings):
    x2 = x.reshape(ROWS, H)
    pos = input_pos.reshape(ROWS).astype(jnp.int32)
    run = functools.partial(
        pl.kernel,
        out_type=jax.ShapeDtypeStruct((ROWS, H), jnp.float32),
        mesh=plsc.VectorSubcoreMesh(core_axis_name="c", subcore_axis_name="s"),
        scratch_types=[
            pltpu.VMEM((ROWS_PER_W,), jnp.int32),
            pltpu.VMEM((NB, CHUNK, H), jnp.float32),
            pltpu.VMEM((NB, CHUNK, H), jnp.float32),
        ] + [pltpu.SemaphoreType.DMA] * (3 * NB),
    )(_sc_body)
    out = run(x2, pos, position_embeddings)
    return out.reshape(B, S, H)
